# baseline (device time: 10613 ns/iter reference)
import jax
import jax.numpy as jnp
from jax import lax
from jax.experimental import pallas as pl
from jax.experimental.pallas import tpu as pltpu

N_DEV = 8
EPS = 1e-5


def kernel(x, gamma):
    m, n_per = x.shape
    n_global = N_DEV * n_per
    mh = m // 2

    def body(
        x_hbm,
        g_hbm,
        out_hbm,
        xv,
        gv,
        ov,
        comm_ref,
        send_sems,
        recv_sems,
        cp_sems,
    ):
        my = lax.axis_index("i")

        barrier_sem = pltpu.get_barrier_semaphore()
        for k in range(1, N_DEV):
            tgt = lax.rem(my + k, N_DEV)
            pl.semaphore_signal(
                barrier_sem,
                inc=1,
                device_id=(tgt,),
                device_id_type=pl.DeviceIdType.MESH,
            )

        cp_x0 = pltpu.make_async_copy(
            x_hbm.at[pl.ds(0, mh)], xv.at[pl.ds(0, mh)], cp_sems.at[0]
        )
        cp_x1 = pltpu.make_async_copy(
            x_hbm.at[pl.ds(mh, mh)], xv.at[pl.ds(mh, mh)], cp_sems.at[1]
        )
        cp_g = pltpu.make_async_copy(g_hbm, gv, cp_sems.at[2])
        cp_x0.start()
        cp_x1.start()
        cp_g.start()

        cp_x0.wait()
        xb0 = xv[0:mh, :].astype(jnp.bfloat16)
        part0 = jnp.sum((xb0 * xb0).astype(jnp.float32), axis=1)
        comm_ref[pl.ds(my, 1), 0:mh] = part0.reshape(1, mh)
        cp_x1.wait()
        xb1 = xv[mh:m, :].astype(jnp.bfloat16)
        part1 = jnp.sum((xb1 * xb1).astype(jnp.float32), axis=1)
        comm_ref[pl.ds(my, 1), mh:m] = part1.reshape(1, mh)

        pl.semaphore_wait(barrier_sem, N_DEV - 1)

        sends = []
        for k in range(1, N_DEV):
            tgt = lax.rem(my + k, N_DEV)
            rdma = pltpu.make_async_remote_copy(
                src_ref=comm_ref.at[pl.ds(my, 1)],
                dst_ref=comm_ref.at[pl.ds(my, 1)],
                send_sem=send_sems.at[k - 1],
                recv_sem=recv_sems.at[k - 1],
                device_id=(tgt,),
                device_id_type=pl.DeviceIdType.MESH,
            )
            rdma.start()
            sends.append(rdma)

        cp_g.wait()
        g_row = gv[:].astype(jnp.bfloat16).reshape(1, n_per)
        xg0 = xb0 * g_row
        xg1 = xb1 * g_row

        for k in range(1, N_DEV):
            src = lax.rem(my - k + N_DEV, N_DEV)
            recv = pltpu.make_async_remote_copy(
                src_ref=comm_ref.at[pl.ds(my, 1)],
                dst_ref=comm_ref.at[pl.ds(src, 1)],
                send_sem=send_sems.at[k - 1],
                recv_sem=recv_sems.at[k - 1],
                device_id=(my,),
                device_id_type=pl.DeviceIdType.MESH,
            )
            recv.wait_recv()

        total = jnp.sum(comm_ref[:, :], axis=0).reshape(1, m)
        inv = lax.rsqrt(total / n_global + EPS)
        inv_col = inv.reshape(m, 1).astype(jnp.bfloat16)

        ov[0:mh, :] = xg0 * inv_col[0:mh, :]
        cp_o0 = pltpu.make_async_copy(
            ov.at[pl.ds(0, mh)], out_hbm.at[pl.ds(0, mh)], cp_sems.at[0]
        )
        cp_o0.start()
        ov[mh:m, :] = xg1 * inv_col[mh:m, :]
        cp_o1 = pltpu.make_async_copy(
            ov.at[pl.ds(mh, mh)], out_hbm.at[pl.ds(mh, mh)], cp_sems.at[1]
        )
        cp_o1.start()
        cp_o0.wait()
        cp_o1.wait()

        for rdma in sends:
            rdma.wait_send()

    return pl.pallas_call(
        body,
        out_shape=jax.ShapeDtypeStruct((m, n_per), jnp.bfloat16),
        in_specs=[
            pl.BlockSpec(memory_space=pltpu.MemorySpace.HBM),
            pl.BlockSpec(memory_space=pltpu.MemorySpace.HBM),
        ],
        out_specs=pl.BlockSpec(memory_space=pltpu.MemorySpace.HBM),
        scratch_shapes=[
            pltpu.VMEM((m, n_per), jnp.float32),
            pltpu.VMEM((n_per,), jnp.float32),
            pltpu.VMEM((m, n_per), jnp.bfloat16),
            pltpu.VMEM((N_DEV, m), jnp.float32),
            pltpu.SemaphoreType.DMA((N_DEV - 1,)),
            pltpu.SemaphoreType.DMA((N_DEV - 1,)),
            pltpu.SemaphoreType.DMA((3,)),
        ],
        compiler_params=pltpu.CompilerParams(collective_id=0),
    )(x, gamma)


# device time: 10220 ns/iter; 1.0385x vs baseline; 1.0385x over previous
import jax
import jax.numpy as jnp
from jax import lax
from jax.experimental import pallas as pl
from jax.experimental.pallas import tpu as pltpu

N_DEV = 8
EPS = 1e-5


def kernel(x, gamma):
    m, n_per = x.shape
    n_global = N_DEV * n_per

    def body(x_ref, g_ref, out_ref, comm_ref, send_sems, recv_sems):
        my = lax.axis_index("i")

        barrier_sem = pltpu.get_barrier_semaphore()
        for k in range(1, N_DEV):
            tgt = lax.rem(my + k, N_DEV)
            pl.semaphore_signal(
                barrier_sem,
                inc=1,
                device_id=(tgt,),
                device_id_type=pl.DeviceIdType.MESH,
            )

        xb = x_ref[:, :].astype(jnp.bfloat16)
        part = jnp.sum((xb * xb).astype(jnp.float32), axis=1).reshape(1, m)
        comm_ref[pl.ds(my, 1), :] = part

        pl.semaphore_wait(barrier_sem, N_DEV - 1)

        sends = []
        for k in range(1, N_DEV):
            tgt = lax.rem(my + k, N_DEV)
            rdma = pltpu.make_async_remote_copy(
                src_ref=comm_ref.at[pl.ds(my, 1)],
                dst_ref=comm_ref.at[pl.ds(my, 1)],
                send_sem=send_sems.at[k - 1],
                recv_sem=recv_sems.at[k - 1],
                device_id=(tgt,),
                device_id_type=pl.DeviceIdType.MESH,
            )
            rdma.start()
            sends.append(rdma)

        xg = xb * g_ref[:].astype(jnp.bfloat16).reshape(1, n_per)

        for k in range(1, N_DEV):
            src = lax.rem(my - k + N_DEV, N_DEV)
            recv = pltpu.make_async_remote_copy(
                src_ref=comm_ref.at[pl.ds(my, 1)],
                dst_ref=comm_ref.at[pl.ds(src, 1)],
                send_sem=send_sems.at[k - 1],
                recv_sem=recv_sems.at[k - 1],
                device_id=(my,),
                device_id_type=pl.DeviceIdType.MESH,
            )
            recv.wait_recv()

        total = jnp.sum(comm_ref[:, :], axis=0).reshape(1, m)
        inv = lax.rsqrt(total / n_global + EPS)
        inv_col = inv.reshape(m, 1).astype(jnp.bfloat16)
        out_ref[:, :] = (xg * inv_col).astype(out_ref.dtype)

        for rdma in sends:
            rdma.wait_send()

    return pl.pallas_call(
        body,
        out_shape=jax.ShapeDtypeStruct((m, n_per), jnp.bfloat16),
        in_specs=[
            pl.BlockSpec(memory_space=pltpu.VMEM),
            pl.BlockSpec(memory_space=pltpu.VMEM),
        ],
        out_specs=pl.BlockSpec(memory_space=pltpu.VMEM),
        scratch_shapes=[
            pltpu.VMEM((N_DEV, m), jnp.float32),
            pltpu.SemaphoreType.DMA((N_DEV - 1,)),
            pltpu.SemaphoreType.DMA((N_DEV - 1,)),
        ],
        compiler_params=pltpu.CompilerParams(collective_id=0),
    )(x, gamma)
